# baseline (speedup 1.0000x reference)
import functools

import jax
import jax.numpy as jnp
from jax import lax
from jax.experimental import pallas as pl
from jax.experimental.pallas import tpu as pltpu
from jax.experimental.pallas import tpu_sc as plsc

K = 16
L = 16
B, M, N = 4, 2048, 8192
NC, NS = 2, 16
NW = NC * NS
NCH = N // L
NSC = NCH // L
QPB = M // NW
BM = 128


def _cm_body(c_ref, xt_ref, x2_ref, cm_ref):
    c = c_ref[0]
    xt = xt_ref[0]
    x2 = x2_ref[0]
    cross = lax.dot_general(c, xt, (((1,), (0,)), ((), ())),
                            preferred_element_type=jnp.float32)
    s = cross.reshape(BM, L, NCH) + x2[None, :, :]
    cm_ref[0] = jnp.min(s, axis=1)


@jax.jit
def _chunk_mins(cb, xts, x2):
    return pl.pallas_call(
        _cm_body,
        grid=(B, M // BM),
        in_specs=[
            pl.BlockSpec((1, BM, 3), lambda bi, mi: (bi, mi, 0)),
            pl.BlockSpec((1, 3, N), lambda bi, mi: (bi, 0, 0)),
            pl.BlockSpec((1, L, NCH), lambda bi, mi: (bi, 0, 0)),
        ],
        out_specs=pl.BlockSpec((1, BM, NCH), lambda bi, mi: (bi, mi, 0)),
        out_shape=jax.ShapeDtypeStruct((B, M, NCH), jnp.float32),
    )(cb, xts, x2)


def _merge_sorted(fd, fi, d, pid):
    bd_s, bi_s = plsc.sort_key_val(d, pid)
    bd_r = lax.rev(bd_s, (0,))
    bi_r = lax.rev(bi_s, (0,))
    take = (fd < bd_r) | ((fd == bd_r) & (fi < bi_r))
    md = jnp.where(take, fd, bd_r)
    mi = jnp.where(take, fi, bi_r)
    sd, si = plsc.sort_key_val(md, mi)
    return sd, si


def _sc_body(xt_hbm, yt_hbm, zt_hbm, x2t_hbm, qx_hbm, qy_hbm, qz_hbm, c2_hbm,
             cm_hbm, od_hbm, oi_hbm,
             xv, yv, zv, x2v, qxv, qyv, qzv, c2sv, cmv, odv, oiv):
    wid = lax.axis_index("s") * NC + lax.axis_index("c")
    iota = lax.broadcasted_iota(jnp.int32, (L,), 0)
    inf_v = jnp.full((L,), jnp.inf, jnp.float32)
    zero_i = jnp.zeros((L,), jnp.int32)

    for b in range(B):
        pltpu.sync_copy(xt_hbm.at[b], xv)
        pltpu.sync_copy(yt_hbm.at[b], yv)
        pltpu.sync_copy(zt_hbm.at[b], zv)
        pltpu.sync_copy(x2t_hbm.at[b], x2v)
        qoff0 = wid * (QPB * L)
        qrow = wid * QPB
        pltpu.sync_copy(qx_hbm.at[b, pl.ds(qrow, QPB)], qxv)
        pltpu.sync_copy(qy_hbm.at[b, pl.ds(qrow, QPB)], qyv)
        pltpu.sync_copy(qz_hbm.at[b, pl.ds(qrow, QPB)], qzv)
        pltpu.sync_copy(c2_hbm.at[b, pl.ds(qrow, QPB)], c2sv)
        pltpu.sync_copy(cm_hbm.at[b, pl.ds(qrow * NCH, QPB * NCH)], cmv)

        def q_body(q, _):
            qoff = q * L
            qsel = jnp.full((L,), q, jnp.int32)
            qx = plsc.load_gather(qxv, [qsel])
            qy = plsc.load_gather(qyv, [qsel])
            qz = plsc.load_gather(qzv, [qsel])
            c2 = plsc.load_gather(c2sv, [qsel])
            cmoff = q * NCH

            def pb_body(s, carry):
                rd, ri = carry
                cm = cmv[pl.ds(cmoff + s * L, L)]
                return _merge_sorted(rd, ri, cm, s * L + iota)

            _, cand_i = lax.fori_loop(0, NSC, pb_body, (inf_v, zero_i))

            def pc_body(j, carry):
                fd, fi = carry
                cid = jnp.sum(jnp.where(iota == j, cand_i, 0))
                pid = cid + NCH * iota
                xc = plsc.load_gather(xv, [pid])
                yc = plsc.load_gather(yv, [pid])
                zc = plsc.load_gather(zv, [pid])
                x2c = plsc.load_gather(x2v, [pid])
                d = (c2 + ((qx * xc + qy * yc) + qz * zc)) + x2c
                return _merge_sorted(fd, fi, d, pid)

            fin_d, fin_i = lax.fori_loop(0, K, pc_body, (inf_v, zero_i))
            odv[pl.ds(qoff, L)] = fin_d
            oiv[pl.ds(qoff, L)] = fin_i
            return 0

        lax.fori_loop(0, QPB, q_body, 0)
        pltpu.sync_copy(odv, od_hbm.at[b, pl.ds(qoff0, QPB * L)])
        pltpu.sync_copy(oiv, oi_hbm.at[b, pl.ds(qoff0, QPB * L)])


_sc_knn = functools.partial(
    pl.kernel,
    out_type=[
        jax.ShapeDtypeStruct((B, M * L), jnp.float32),
        jax.ShapeDtypeStruct((B, M * L), jnp.int32),
    ],
    mesh=plsc.VectorSubcoreMesh(
        core_axis_name="c", subcore_axis_name="s",
        num_cores=NC, num_subcores=NS),
    compiler_params=pltpu.CompilerParams(needs_layout_passes=False),
    scratch_types=[
        pltpu.VMEM((N,), jnp.float32),
        pltpu.VMEM((N,), jnp.float32),
        pltpu.VMEM((N,), jnp.float32),
        pltpu.VMEM((N,), jnp.float32),
        pltpu.VMEM((QPB,), jnp.float32),
        pltpu.VMEM((QPB,), jnp.float32),
        pltpu.VMEM((QPB,), jnp.float32),
        pltpu.VMEM((QPB,), jnp.float32),
        pltpu.VMEM((QPB * NCH,), jnp.float32),
        pltpu.VMEM((QPB * L,), jnp.float32),
        pltpu.VMEM((QPB * L,), jnp.int32),
    ],
)(_sc_body)


@jax.jit
def _knn(center_xyz, xyz):
    xyzb = lax.reduce_precision(xyz, exponent_bits=8, mantissa_bits=7)
    cb = lax.reduce_precision(center_xyz, exponent_bits=8, mantissa_bits=7)
    xyzb = -2.0 * xyzb
    x2 = jnp.sum(xyz * xyz, axis=-1)
    xts = xyzb.transpose(0, 2, 1)
    cm = _chunk_mins(cb, xts, x2.reshape(B, L, NCH))
    xt = xyzb[..., 0]
    yt = xyzb[..., 1]
    zt = xyzb[..., 2]
    c2 = jnp.sum(center_xyz * center_xyz, axis=-1)
    qx = cb[..., 0]
    qy = cb[..., 1]
    qz = cb[..., 2]
    od, oi = _sc_knn(xt, yt, zt, x2, qx, qy, qz, c2,
                     cm.reshape(B, M * NCH))
    return od.reshape(B, M, K), oi.reshape(B, M, K)


def kernel(center_xyz, xyz, points):
    del points
    return tuple(_knn(center_xyz, xyz))


# speedup vs baseline: 1.1405x; 1.1405x over previous
import functools

import jax
import jax.numpy as jnp
from jax import lax
from jax.experimental import pallas as pl
from jax.experimental.pallas import tpu as pltpu
from jax.experimental.pallas import tpu_sc as plsc

K = 16
L = 16
B, M, N = 4, 2048, 8192
NC, NS = 2, 16
NW = NC * NS
NCH = N // L
NSC = NCH // L
QPB = M // NW
BM = 128


def _cm_body(c_ref, xt_ref, x2_ref, cm_ref):
    c = c_ref[0]
    xt = xt_ref[0]
    x2 = x2_ref[0]
    s = None
    for j in range(L):
        xj = xt[:, j * NCH:(j + 1) * NCH]
        cj = lax.dot_general(c, xj, (((1,), (0,)), ((), ())),
                             preferred_element_type=jnp.float32)
        sj = cj + x2[j:j + 1, :]
        s = sj if s is None else jnp.minimum(s, sj)
    cm_ref[0] = s


@jax.jit
def _chunk_mins(cb, xts, x2):
    return pl.pallas_call(
        _cm_body,
        grid=(B, M // BM),
        in_specs=[
            pl.BlockSpec((1, BM, 3), lambda bi, mi: (bi, mi, 0)),
            pl.BlockSpec((1, 3, N), lambda bi, mi: (bi, 0, 0)),
            pl.BlockSpec((1, L, NCH), lambda bi, mi: (bi, 0, 0)),
        ],
        out_specs=pl.BlockSpec((1, BM, NCH), lambda bi, mi: (bi, mi, 0)),
        out_shape=jax.ShapeDtypeStruct((B, M, NCH), jnp.float32),
    )(cb, xts, x2)


def _merge_sorted(fd, fi, d, pid):
    bd_s, bi_s = plsc.sort_key_val(d, pid)
    bd_r = lax.rev(bd_s, (0,))
    bi_r = lax.rev(bi_s, (0,))
    take = (fd < bd_r) | ((fd == bd_r) & (fi < bi_r))
    md = jnp.where(take, fd, bd_r)
    mi = jnp.where(take, fi, bi_r)
    sd, si = plsc.sort_key_val(md, mi)
    return sd, si


def _sc_body(xt_hbm, yt_hbm, zt_hbm, x2t_hbm, qx_hbm, qy_hbm, qz_hbm, c2_hbm,
             cm_hbm, od_hbm, oi_hbm,
             xv, yv, zv, x2v, qxv, qyv, qzv, c2sv, cmv, odv, oiv):
    wid = lax.axis_index("s") * NC + lax.axis_index("c")
    iota = lax.broadcasted_iota(jnp.int32, (L,), 0)
    inf_v = jnp.full((L,), jnp.inf, jnp.float32)
    zero_i = jnp.zeros((L,), jnp.int32)

    for b in range(B):
        pltpu.sync_copy(xt_hbm.at[b], xv)
        pltpu.sync_copy(yt_hbm.at[b], yv)
        pltpu.sync_copy(zt_hbm.at[b], zv)
        pltpu.sync_copy(x2t_hbm.at[b], x2v)
        qoff0 = wid * (QPB * L)
        qrow = wid * QPB
        pltpu.sync_copy(qx_hbm.at[b, pl.ds(qrow, QPB)], qxv)
        pltpu.sync_copy(qy_hbm.at[b, pl.ds(qrow, QPB)], qyv)
        pltpu.sync_copy(qz_hbm.at[b, pl.ds(qrow, QPB)], qzv)
        pltpu.sync_copy(c2_hbm.at[b, pl.ds(qrow, QPB)], c2sv)
        pltpu.sync_copy(cm_hbm.at[b, pl.ds(qrow * NCH, QPB * NCH)], cmv)

        def q_body(q, _):
            qoff = q * L
            qsel = jnp.full((L,), q, jnp.int32)
            qx = plsc.load_gather(qxv, [qsel])
            qy = plsc.load_gather(qyv, [qsel])
            qz = plsc.load_gather(qzv, [qsel])
            c2 = plsc.load_gather(c2sv, [qsel])
            cmoff = q * NCH

            def pb_body(s, carry):
                rd, ri = carry
                cm = cmv[pl.ds(cmoff + s * L, L)]
                return _merge_sorted(rd, ri, cm, s * L + iota)

            _, cand_i = lax.fori_loop(0, NSC, pb_body, (inf_v, zero_i))

            def pc_body(j, carry):
                fd, fi = carry
                cid = jnp.sum(jnp.where(iota == j, cand_i, 0))
                pid = cid + NCH * iota
                xc = plsc.load_gather(xv, [pid])
                yc = plsc.load_gather(yv, [pid])
                zc = plsc.load_gather(zv, [pid])
                x2c = plsc.load_gather(x2v, [pid])
                d = (c2 + ((qx * xc + qy * yc) + qz * zc)) + x2c
                return _merge_sorted(fd, fi, d, pid)

            fin_d, fin_i = lax.fori_loop(0, K, pc_body, (inf_v, zero_i))
            odv[pl.ds(qoff, L)] = fin_d
            oiv[pl.ds(qoff, L)] = fin_i
            return 0

        lax.fori_loop(0, QPB, q_body, 0)
        pltpu.sync_copy(odv, od_hbm.at[b, pl.ds(qoff0, QPB * L)])
        pltpu.sync_copy(oiv, oi_hbm.at[b, pl.ds(qoff0, QPB * L)])


_sc_knn = functools.partial(
    pl.kernel,
    out_type=[
        jax.ShapeDtypeStruct((B, M * L), jnp.float32),
        jax.ShapeDtypeStruct((B, M * L), jnp.int32),
    ],
    mesh=plsc.VectorSubcoreMesh(
        core_axis_name="c", subcore_axis_name="s",
        num_cores=NC, num_subcores=NS),
    compiler_params=pltpu.CompilerParams(needs_layout_passes=False),
    scratch_types=[
        pltpu.VMEM((N,), jnp.float32),
        pltpu.VMEM((N,), jnp.float32),
        pltpu.VMEM((N,), jnp.float32),
        pltpu.VMEM((N,), jnp.float32),
        pltpu.VMEM((QPB,), jnp.float32),
        pltpu.VMEM((QPB,), jnp.float32),
        pltpu.VMEM((QPB,), jnp.float32),
        pltpu.VMEM((QPB,), jnp.float32),
        pltpu.VMEM((QPB * NCH,), jnp.float32),
        pltpu.VMEM((QPB * L,), jnp.float32),
        pltpu.VMEM((QPB * L,), jnp.int32),
    ],
)(_sc_body)


@jax.jit
def _knn(center_xyz, xyz):
    xyzb = lax.reduce_precision(xyz, exponent_bits=8, mantissa_bits=7)
    cb = lax.reduce_precision(center_xyz, exponent_bits=8, mantissa_bits=7)
    xyzb = -2.0 * xyzb
    x2 = jnp.sum(xyz * xyz, axis=-1)
    xts = xyzb.transpose(0, 2, 1)
    cm = _chunk_mins(cb, xts, x2.reshape(B, L, NCH))
    xt = xyzb[..., 0]
    yt = xyzb[..., 1]
    zt = xyzb[..., 2]
    c2 = jnp.sum(center_xyz * center_xyz, axis=-1)
    qx = cb[..., 0]
    qy = cb[..., 1]
    qz = cb[..., 2]
    od, oi = _sc_knn(xt, yt, zt, x2, qx, qy, qz, c2,
                     cm.reshape(B, M * NCH))
    return od.reshape(B, M, K), oi.reshape(B, M, K)


def kernel(center_xyz, xyz, points):
    del points
    return tuple(_knn(center_xyz, xyz))
